# baseline (speedup 1.0000x reference)
import jax
import jax.numpy as jnp
from jax.experimental import pallas as pl

TOTAL = 16384 * 1000
BLOCK = 1024 * 1000


def _fill_block(o_ref):
    o_ref[...] = jnp.zeros((BLOCK,), jnp.float32)


def kernel(x):
    nb = TOTAL // BLOCK
    out = pl.pallas_call(
        _fill_block,
        grid=(nb,),
        out_specs=pl.BlockSpec((BLOCK,), lambda i: (i,)),
        out_shape=jax.ShapeDtypeStruct((TOTAL,), jnp.float32),
    )()
    return out.reshape(16384, 1000)


# speedup vs baseline: 1.9044x; 1.9044x over previous
import jax
import jax.numpy as jnp
from jax.experimental import pallas as pl
from jax.experimental.pallas import tpu as pltpu

NUM_CLASSES_ = 1000
N_ = 16384
BLOCK_ROWS = 2048
Q_ = 4
RQ_ = BLOCK_ROWS // Q_
NB_ = N_ // BLOCK_ROWS


def _copy(scratch, o_hbm, sems, par, step, q):
    return pltpu.make_async_copy(
        scratch.at[par, pl.ds(q * RQ_, RQ_), :],
        o_hbm.at[pl.ds(step * BLOCK_ROWS + q * RQ_, RQ_), :],
        sems.at[par, q],
    )


def _onehot_block(x_ref, o_hbm, scratch, sems):
    i = pl.program_id(0)
    par = jax.lax.rem(i, 2)

    @pl.when(i >= 2)
    def _wait_prev():
        for q in range(Q_):
            _copy(scratch, o_hbm, sems, par, i - 2, q).wait()

    xb = x_ref[0, 0, :]
    col = jax.lax.broadcasted_iota(jnp.int32, (BLOCK_ROWS, NUM_CLASSES_), 1)
    scratch[par] = (xb[:, None] == col).astype(jnp.float32)

    for q in range(Q_):
        _copy(scratch, o_hbm, sems, par, i, q).start()

    @pl.when(i == NB_ - 1)
    def _drain():
        for q in range(Q_):
            _copy(scratch, o_hbm, sems, 1 - par, i - 1, q).wait()
        for q in range(Q_):
            _copy(scratch, o_hbm, sems, par, i, q).wait()


def kernel(x):
    x3 = x.astype(jnp.int32).reshape(NB_, 1, BLOCK_ROWS)
    out = pl.pallas_call(
        _onehot_block,
        grid=(NB_,),
        in_specs=[pl.BlockSpec((1, 1, BLOCK_ROWS), lambda i: (i, 0, 0))],
        out_specs=pl.BlockSpec(memory_space=pl.ANY),
        out_shape=jax.ShapeDtypeStruct((N_, NUM_CLASSES_), jnp.float32),
        scratch_shapes=[
            pltpu.VMEM((2, BLOCK_ROWS, NUM_CLASSES_), jnp.float32),
            pltpu.SemaphoreType.DMA((2, Q_)),
        ],
    )(x3)
    return out
